# baseline (speedup 1.0000x reference)
import functools

import jax
import jax.numpy as jnp
from jax import lax
from jax.experimental import pallas as pl
from jax.experimental.pallas import tpu as pltpu
from jax.experimental.pallas import tpu_sc as plsc

N_ROWS = 320000
D = 128
NUM_SEGMENTS = 10000
NC = 2
NS = 16
NW = NC * NS
SEG_PER_W = 320
S_PAD = NW * SEG_PER_W
CHUNK = 160
LANES = 16
NVREG = D // LANES
BOUNDS_PAD = 48
IDX_PAD = CHUNK + LANES


def _sc_body(proc_hbm, idx_hbm, bounds_hbm, m_hbm, s_hbm,
             bnd_v, rows_a, rows_b, idx_a, idx_bb, m_st, s_st,
             acc_m, acc_s, sem_ra, sem_rb, sem_ia, sem_ib):
    cid = lax.axis_index("c")
    sid = lax.axis_index("s")
    wid = sid * NC + cid
    seg_lo = pl.multiple_of(wid * SEG_PER_W, 8)

    neg_inf_v = jnp.full((LANES,), -jnp.inf, jnp.float32)
    zero_v = jnp.zeros((LANES,), jnp.float32)

    pltpu.sync_copy(bounds_hbm, bnd_v)
    bnd_vec = bnd_v[pl.ds(wid, LANES)]
    row_lo = bnd_vec[0]
    row_hi = bnd_vec[1]

    def init_body(i, _):
        for j in range(NVREG):
            m_st[pl.ds(i * D + j * LANES, LANES)] = neg_inf_v
            s_st[pl.ds(i * D + j * LANES, LANES)] = zero_v
        return 0
    lax.fori_loop(0, SEG_PER_W, init_body, 0)

    def load_acc():
        m = tuple(acc_m[pl.ds(j * LANES, LANES)] for j in range(NVREG))
        s = tuple(acc_s[pl.ds(j * LANES, LANES)] for j in range(NVREG))
        return m, s

    def store_acc(m, s):
        for j in range(NVREG):
            acc_m[pl.ds(j * LANES, LANES)] = m[j]
            acc_s[pl.ds(j * LANES, LANES)] = s[j]

    def init_acc():
        for j in range(NVREG):
            acc_m[pl.ds(j * LANES, LANES)] = neg_inf_v
            acc_s[pl.ds(j * LANES, LANES)] = zero_v

    def flush(g_cur):
        off = (g_cur - seg_lo) * D
        for j in range(NVREG):
            m_st[pl.ds(off + j * LANES, LANES)] = acc_m[pl.ds(j * LANES, LANES)]
            s_st[pl.ds(off + j * LANES, LANES)] = acc_s[pl.ds(j * LANES, LANES)]

    def dma_handles(c, rows_buf, idx_buf, semr, semi):
        base = pl.multiple_of(c * CHUNK, 8)
        hr = pltpu.make_async_copy(
            proc_hbm.at[pl.ds(base * D, CHUNK * D)], rows_buf, semr)
        hi = pltpu.make_async_copy(
            idx_hbm.at[pl.ds(base, CHUNK)], idx_buf.at[pl.ds(0, CHUNK)], semi)
        return hr, hi

    def start_dma(c, rows_buf, idx_buf, semr, semi):
        hr, hi = dma_handles(c, rows_buf, idx_buf, semr, semi)
        hr.start()
        hi.start()

    c0 = row_lo // CHUNK
    c1 = (row_hi + CHUNK - 1) // CHUNK

    @pl.when(c1 > c0)
    def _():
        start_dma(c0, rows_a, idx_a, sem_ra, sem_ia)

    def process(c, rows_v, idx_v, semr, semi, n_rows, n_idx, n_semr, n_semi,
                carry):
        hr, hi = dma_handles(c, rows_v, idx_v, semr, semi)
        hr.wait()
        hi.wait()

        @pl.when(c + 1 < c1)
        def _():
            start_dma(c + 1, n_rows, n_idx, n_semr, n_semi)

        base = pl.multiple_of(c * CHUNK, 8)
        i_lo = lax.max(row_lo - base, 0)
        i_hi = lax.min(row_hi - base, CHUNK)

        def row_body(i, g_cur):
            g = idx_v[pl.ds(i, LANES)][0]
            changed = g != g_cur

            @pl.when(jnp.logical_and(changed, g_cur >= 0))
            def _():
                flush(g_cur)

            @pl.when(changed)
            def _():
                init_acc()

            m, s = load_acc()
            new_m = []
            new_s = []
            for j in range(NVREG):
                x = rows_v[pl.ds(i * D + j * LANES, LANES)]
                m2 = jnp.maximum(m[j], x)
                s2 = s[j] * jnp.exp(m[j] - m2) + jnp.exp(x - m2)
                new_m.append(m2)
                new_s.append(s2)
            store_acc(tuple(new_m), tuple(new_s))
            return g

        def fast_group(gl, g_cur):
            m, s = load_acc()
            for q in range(4):
                base_off = (gl * LANES + q * 4) * D
                new_m = []
                new_s = []
                for j in range(NVREG):
                    o = base_off + j * LANES
                    x0 = rows_v[pl.ds(o, LANES)]
                    x1 = rows_v[pl.ds(o + D, LANES)]
                    x2 = rows_v[pl.ds(o + 2 * D, LANES)]
                    x3 = rows_v[pl.ds(o + 3 * D, LANES)]
                    mx = jnp.maximum(jnp.maximum(x0, x1),
                                     jnp.maximum(x2, x3))
                    m2 = jnp.maximum(m[j], mx)
                    e = (jnp.exp(x0 - m2) + jnp.exp(x1 - m2)) + \
                        (jnp.exp(x2 - m2) + jnp.exp(x3 - m2))
                    s2 = s[j] * jnp.exp(m[j] - m2) + e
                    new_m.append(m2)
                    new_s.append(s2)
                m = tuple(new_m)
                s = tuple(new_s)
            store_acc(m, s)
            return g_cur

        gl0 = i_lo // LANES
        gl1 = (i_hi + LANES - 1) // LANES

        def group_body(gl, g_cur):
            j_lo = lax.max(i_lo - gl * LANES, 0)
            j_hi = lax.min(i_hi - gl * LANES, LANES)
            gvec = idx_v[pl.ds(gl * LANES, LANES)]
            full = jnp.logical_and(j_lo == 0, j_hi == LANES)
            same = jnp.logical_and(gvec[0] == g_cur, gvec[LANES - 1] == g_cur)
            fast = jnp.logical_and(full, same)
            return lax.cond(
                fast,
                lambda cr: fast_group(gl, cr),
                lambda cr: lax.fori_loop(gl * LANES + j_lo,
                                         gl * LANES + j_hi, row_body, cr),
                g_cur)

        return lax.fori_loop(gl0, gl1, group_body, carry)

    def chunk_body(c, carry):
        even = ((c - c0) % 2) == 0
        return lax.cond(
            even,
            lambda cr: process(c, rows_a, idx_a, sem_ra, sem_ia,
                               rows_b, idx_bb, sem_rb, sem_ib, cr),
            lambda cr: process(c, rows_b, idx_bb, sem_rb, sem_ib,
                               rows_a, idx_a, sem_ra, sem_ia, cr),
            carry)

    g_cur = lax.fori_loop(c0, c1, chunk_body, jnp.int32(-1))

    @pl.when(g_cur >= 0)
    def _():
        flush(g_cur)

    out_off = pl.multiple_of(seg_lo * D, 8)
    pltpu.sync_copy(m_st, m_hbm.at[pl.ds(out_off, SEG_PER_W * D)])
    pltpu.sync_copy(s_st, s_hbm.at[pl.ds(out_off, SEG_PER_W * D)])


_sc_call = functools.partial(
    pl.kernel,
    out_type=(
        jax.ShapeDtypeStruct((S_PAD * D,), jnp.float32),
        jax.ShapeDtypeStruct((S_PAD * D,), jnp.float32),
    ),
    mesh=plsc.VectorSubcoreMesh(
        core_axis_name="c", subcore_axis_name="s",
        num_cores=NC, num_subcores=NS,
    ),
    scratch_types=[
        pltpu.VMEM((BOUNDS_PAD,), jnp.int32),
        pltpu.VMEM((CHUNK * D,), jnp.float32),
        pltpu.VMEM((CHUNK * D,), jnp.float32),
        pltpu.VMEM((IDX_PAD,), jnp.int32),
        pltpu.VMEM((IDX_PAD,), jnp.int32),
        pltpu.VMEM((SEG_PER_W * D,), jnp.float32),
        pltpu.VMEM((SEG_PER_W * D,), jnp.float32),
        pltpu.VMEM((D,), jnp.float32),
        pltpu.VMEM((D,), jnp.float32),
        pltpu.SemaphoreType.DMA,
        pltpu.SemaphoreType.DMA,
        pltpu.SemaphoreType.DMA,
        pltpu.SemaphoreType.DMA,
    ],
)(_sc_body)


def _finalize_body(m_ref, s_ref, out_ref):
    m = m_ref[0:NUM_SEGMENTS, :]
    s = s_ref[0:NUM_SEGMENTS, :]
    out = jnp.log(s) + m
    gmax = jnp.max(out)
    t = jnp.sum(jnp.exp(out - gmax))
    z = jnp.log(t) + gmax
    out_ref[...] = out - z


_finalize_call = pl.pallas_call(
    _finalize_body,
    out_shape=jax.ShapeDtypeStruct((NUM_SEGMENTS, D), jnp.float32),
)


@jax.jit
def kernel(proc, idx_b):
    seg_starts = jnp.arange(NW + 1, dtype=jnp.int32) * SEG_PER_W
    bounds = jnp.searchsorted(idx_b, seg_starts, side="left").astype(jnp.int32)
    bounds = jnp.pad(bounds, (0, BOUNDS_PAD - (NW + 1)))
    m_all, s_all = _sc_call(proc.reshape(N_ROWS * D), idx_b, bounds)
    return _finalize_call(m_all.reshape(S_PAD, D), s_all.reshape(S_PAD, D))
